# baseline (speedup 1.0000x reference)
import functools

import jax
import jax.numpy as jnp
from jax import lax
from jax.experimental import pallas as pl
from jax.experimental.pallas import tpu as pltpu
from jax.experimental.pallas import tpu_sc as plsc

D = 1024
NW = 32
CHUNK = 16
NBUF = 6
LOOKAHEAD = 3


def _normalize_rows_body(t_ref, o_ref):
    x = t_ref[...]
    s = jnp.sum(x * x, axis=1, keepdims=True)
    norm = jnp.sqrt(s)
    o_ref[...] = x * (1.0 / jnp.maximum(norm, 1e-12))


def _normalize_table(table):
    rows, d = table.shape
    blk = 2048
    return pl.pallas_call(
        _normalize_rows_body,
        grid=(rows // blk,),
        in_specs=[pl.BlockSpec((blk, d), lambda i: (i, 0))],
        out_specs=pl.BlockSpec((blk, d), lambda i: (i, 0)),
        out_shape=jax.ShapeDtypeStruct((rows, d), table.dtype),
    )(table)


def _make_sc_gather(n_rows_total):
    n_per_w = n_rows_total // NW
    n_chunks = n_per_w // CHUNK
    assert n_chunks >= 3 * NBUF
    assert LOOKAHEAD <= NBUF
    n_full = (n_chunks // NBUF) * NBUF
    mesh = plsc.VectorSubcoreMesh(core_axis_name="c", subcore_axis_name="s")

    @functools.partial(
        pl.kernel,
        mesh=mesh,
        out_type=jax.ShapeDtypeStruct((n_rows_total, D), jnp.float32),
        scratch_types=[
            pltpu.VMEM((n_chunks, CHUNK), jnp.int32),
            pltpu.VMEM((NBUF, CHUNK, D), jnp.float32),
        ] + [pltpu.SemaphoreType.DMA] * (2 * NBUF),
    )
    def gather_kernel(table_hbm, idx_hbm, out_hbm, idx_v, buf, *sems):
        gs = sems[:NBUF]
        ss = sems[NBUF:]
        wid = lax.axis_index("s") * 2 + lax.axis_index("c")
        pltpu.sync_copy(idx_hbm.at[wid], idx_v)
        base = wid * n_per_w

        def start_gather(jf, b):
            pltpu.async_copy(table_hbm.at[idx_v.at[jf]], buf.at[b], gs[b])

        def wait_gather(j, b):
            pltpu.make_async_copy(table_hbm.at[idx_v.at[j]], buf.at[b],
                                  gs[b]).wait()

        def start_scatter(j, b):
            pltpu.async_copy(buf.at[b],
                             out_hbm.at[pl.ds(base + j * CHUNK, CHUNK)], ss[b])

        def wait_scatter(j, b):
            pltpu.make_async_copy(buf.at[b],
                                  out_hbm.at[pl.ds(base + j * CHUNK, CHUNK)],
                                  ss[b]).wait()

        def step(j, b):
            wait_gather(j, b)
            start_scatter(j, b)
            jf = j + LOOKAHEAD
            do_feed = (jf < n_chunks) if isinstance(j, int) else True
            if do_feed:
                bf = (b + LOOKAHEAD) % NBUF
                js = jf - NBUF
                do_drain = (js >= 0) if isinstance(j, int) else True
                if do_drain:
                    wait_scatter(js, bf)
                start_gather(jf, bf)

        for j in range(LOOKAHEAD):
            start_gather(j, j % NBUF)
        for j in range(NBUF):
            step(j, j)

        def body(kk, carry):
            for b in range(NBUF):
                step(kk * NBUF + b, b)
            return carry

        lax.fori_loop(1, n_full // NBUF - 1, body, 0)

        for j in range(n_full - NBUF, n_chunks):
            step(j, j % NBUF)
        for j in range(n_chunks - NBUF, n_chunks):
            wait_scatter(j, j % NBUF)

    return gather_kernel


def kernel(positions, positional_embeddings):
    b = positions.size
    n_per_w = b // NW
    n_chunks = n_per_w // CHUNK
    norm_table = _normalize_table(positional_embeddings)
    idx = positions.reshape(NW, n_chunks, CHUNK).astype(jnp.int32)
    out = _make_sc_gather(b)(norm_table, idx)
    return out.reshape(positions.shape + (D,))
